# baseline (speedup 1.0000x reference)
import functools

import jax
import jax.numpy as jnp
from jax.experimental import pallas as pl


def _sage_kernel(x_ref, xo_ref, adj_ref, w_ref, b_ref, out_ref, *, block_m, d_in):
    i = pl.program_id(0)
    af = adj_ref[...].astype(jnp.bfloat16)
    sfull = jnp.dot(af, xo_ref[...], preferred_element_type=jnp.float32)
    s = sfull[:, :d_in]
    deg = sfull[:, d_in:d_in + 1]
    x1 = s / deg
    xr = x_ref[pl.ds(i * block_m, block_m), :]
    b = b_ref[...]
    for k in range(out_ref.shape[0]):
        w1 = w_ref[k, :d_in, :]
        w2 = w_ref[k, d_in:, :]
        out_ref[k] = (
            jnp.dot(x1, w1, preferred_element_type=jnp.float32)
            + jnp.dot(xr, w2, preferred_element_type=jnp.float32)
            + b
        )


def kernel(g, x, adj, W, b):
    n, d_in = x.shape
    k3, two_d, d_out = W.shape
    block_m = 512
    grid = (n // block_m,)
    xo = jnp.concatenate(
        [x, jnp.ones((n, 1), dtype=x.dtype)], axis=1
    ).astype(jnp.bfloat16)
    body = functools.partial(_sage_kernel, block_m=block_m, d_in=d_in)
    out = pl.pallas_call(
        body,
        grid=grid,
        in_specs=[
            pl.BlockSpec((n, d_in), lambda i: (0, 0)),
            pl.BlockSpec((n, d_in + 1), lambda i: (0, 0)),
            pl.BlockSpec((block_m, n), lambda i: (i, 0)),
            pl.BlockSpec((k3, two_d, d_out), lambda i: (0, 0, 0)),
            pl.BlockSpec((d_out,), lambda i: (0,)),
        ],
        out_specs=pl.BlockSpec((k3, block_m, d_out), lambda i: (0, i, 0)),
        out_shape=jax.ShapeDtypeStruct((k3, n, d_out), jnp.float32),
    )(x, xo, adj, W, b)
    return out


# speedup vs baseline: 1.0174x; 1.0174x over previous
import functools

import jax
import jax.numpy as jnp
from jax.experimental import pallas as pl
from jax.experimental.pallas import tpu as pltpu


def _sage_kernel(x_ref, xo_ref, adj_ref, w_ref, b_ref, out_ref, *, block_m, d_in):
    i = pl.program_id(0)
    af = adj_ref[...].astype(jnp.bfloat16)
    sfull = jnp.dot(af, xo_ref[...], preferred_element_type=jnp.float32)
    s = sfull[:, :d_in]
    deg = sfull[:, d_in:d_in + 1]
    x1 = s / deg
    xr = x_ref[pl.ds(i * block_m, block_m), :]
    b = b_ref[...]
    for k in range(out_ref.shape[0]):
        w1 = w_ref[k, :d_in, :]
        w2 = w_ref[k, d_in:, :]
        out_ref[k] = (
            jnp.dot(x1, w1, preferred_element_type=jnp.float32)
            + jnp.dot(xr, w2, preferred_element_type=jnp.float32)
            + b
        )


def kernel(g, x, adj, W, b):
    n, d_in = x.shape
    k3, two_d, d_out = W.shape
    block_m = 512
    grid = (n // block_m,)
    xo = jnp.concatenate(
        [x, jnp.ones((n, 1), dtype=x.dtype)], axis=1
    ).astype(jnp.bfloat16)
    body = functools.partial(_sage_kernel, block_m=block_m, d_in=d_in)
    out = pl.pallas_call(
        body,
        grid=grid,
        in_specs=[
            pl.BlockSpec((n, d_in), lambda i: (0, 0)),
            pl.BlockSpec((n, d_in + 1), lambda i: (0, 0)),
            pl.BlockSpec((block_m, n), lambda i: (i, 0)),
            pl.BlockSpec((k3, two_d, d_out), lambda i: (0, 0, 0)),
            pl.BlockSpec((d_out,), lambda i: (0,)),
        ],
        out_specs=pl.BlockSpec((k3, block_m, d_out), lambda i: (0, i, 0)),
        out_shape=jax.ShapeDtypeStruct((k3, n, d_out), jnp.float32),
        compiler_params=pltpu.CompilerParams(
            dimension_semantics=("parallel",),
        ),
    )(x, xo, adj, W, b)
    return out
